# baseline (speedup 1.0000x reference)
import functools

import jax
import jax.numpy as jnp
from jax import lax
from jax.experimental import pallas as pl
from jax.experimental.pallas import tpu as pltpu
from jax.experimental.pallas import tpu_sc as plsc

N = 10000
E = 320000
RBF = 64
DN = 128
DE = 128
DA = 64

NC = 2
NS = 16
NW = NC * NS
RPW = 384
NFULL = N // RPW
NREM = N - NFULL * RPW
L = 16
S_PAD = 16

@functools.cache
def _make_sc_gather():
    mesh = plsc.VectorSubcoreMesh(core_axis_name="c", subcore_axis_name="s")

    @functools.partial(
        pl.kernel,
        mesh=mesh,
        out_type=[
            jax.ShapeDtypeStruct((N, DN), jnp.float32),
            jax.ShapeDtypeStruct((DA,), jnp.float32),
        ],
        scratch_types=[
            pltpu.VMEM((RPW,), jnp.int32),
            pltpu.VMEM((RPW, DN), jnp.float32),
            pltpu.VMEM_SHARED((95, DN), jnp.float32),
            pltpu.VMEM((S_PAD,), jnp.int32),
            pltpu.VMEM((DA,), jnp.float32),
            pltpu.VMEM((100 * DA,), jnp.float32),
            pltpu.SemaphoreType.DMA,
            pltpu.SemaphoreType.DMA,
        ],
        compiler_params=pltpu.CompilerParams(needs_layout_passes=False),
    )
    def _sc_gather(node_table_hbm, node_idx_hbm, state_table_hbm,
                   state_idx_hbm, node_out_hbm, state_out_hbm,
                   idx_v, rows_v, tab_v, sidx_v, srow_v, stab_v, sem, sem2):
        wid = lax.axis_index("s") * NC + lax.axis_index("c")
        base = wid * RPW
        @pl.when(lax.axis_index("s") == 0)
        def _():
            pltpu.sync_copy(node_table_hbm, tab_v)
        plsc.subcore_barrier()

        @pl.when(wid < NFULL)
        def _():
            pltpu.sync_copy(node_idx_hbm.at[pl.ds(base, RPW)], idx_v)
            gathers = [
                pltpu.async_copy(tab_v.at[idx_v.at[pl.ds(j * 128, 128)]],
                                 rows_v.at[pl.ds(j * 128, 128)], sem2)
                for j in range(RPW // 128)
            ]
            for g in gathers:
                g.wait()
            pltpu.sync_copy(rows_v, node_out_hbm.at[pl.ds(base, RPW)])

        @pl.when(wid == NFULL)
        def _():
            pltpu.sync_copy(node_idx_hbm.at[pl.ds(NFULL * RPW, NREM)],
                            idx_v.at[pl.ds(0, NREM)])
            pltpu.async_copy(tab_v.at[idx_v.at[pl.ds(0, NREM)]],
                             rows_v.at[pl.ds(0, NREM)], sem2).wait()
            pltpu.sync_copy(rows_v.at[pl.ds(0, NREM)],
                            node_out_hbm.at[pl.ds(NFULL * RPW, NREM)])

        @pl.when(wid == NFULL + 1)
        def _():
            lane = lax.broadcasted_iota(jnp.int32, (L,), 0)
            scp = pltpu.async_copy(state_table_hbm, stab_v, sem)
            pltpu.sync_copy(state_idx_hbm, sidx_v)
            scp.wait()
            srow = sidx_v[...]
            for k in range(DA // L):
                vals = plsc.load_gather(stab_v, [(lane + (k * L)) * 100 + srow])
                srow_v[pl.ds(k * L, L)] = vals
            pltpu.sync_copy(srow_v, state_out_hbm)

    return _sc_gather


BE = 32000


def _mlp_body(xt_ref, w_ref, b_ref, o_ref):
    acc = jax.lax.dot_general(
        xt_ref[...], w_ref[...],
        dimension_numbers=(((0,), (0,)), ((), ())),
        preferred_element_type=jnp.float32,
    )
    acc = acc + b_ref[...]
    o_ref[...] = acc * jax.nn.sigmoid(acc)


def _edge_mlp(edge_attr_t, edge_W, edge_b2d):
    return pl.pallas_call(
        _mlp_body,
        grid=(E // BE,),
        in_specs=[
            pl.BlockSpec((RBF, BE), lambda i: (0, i)),
            pl.BlockSpec((RBF, DE), lambda i: (0, 0)),
            pl.BlockSpec((1, DE), lambda i: (0, 0)),
        ],
        out_specs=pl.BlockSpec((BE, DE), lambda i: (i, 0)),
        out_shape=jax.ShapeDtypeStruct((E, DE), jnp.float32),
        compiler_params=pltpu.CompilerParams(
            dimension_semantics=("parallel",),
            fuse_transposed_lhs_in_matmul=True,
        ),
    )(edge_attr_t, edge_W, edge_b2d)


def kernel(node_attr, edge_attr, state_attr, node_table, edge_W, edge_b, state_table):
    state_idx = jnp.broadcast_to(state_attr.astype(jnp.int32), (S_PAD,))

    node_feat, state_row = _make_sc_gather()(
        node_table, node_attr.astype(jnp.int32), state_table.T.reshape(-1),
        state_idx)
    edge_feat = _edge_mlp(edge_attr.T, edge_W, edge_b.reshape(1, DE))

    return (node_feat, edge_feat, state_row.reshape(1, DA))
